# Initial kernel scaffold; baseline (speedup 1.0000x reference)
import jax
import jax.numpy as jnp
from jax.experimental import pallas as pl


def kernel(edge_index, etypes, inv, ent_type_feat_emb, rel_head_emb, rel_tail_emb, res_ent_emb, W1, b1, W2, b2):
    raise NotImplementedError("write your pallas kernel here")


# speedup vs baseline: 1.7649x; 1.7649x over previous
import functools

import jax
import jax.numpy as jnp
from jax import lax
from jax.experimental import pallas as pl
from jax.experimental.pallas import tpu as pltpu
from jax.experimental.pallas import tpu_sc as plsc

N_NODES = 10000
N_EDGES = 160000
HID = 256
NREL = 500
NRES = 50000

NC = 2
NS = 16
HHALF = HID // 2

EDGE_PER_SUB = N_EDGES // NS
EBATCH = 80
NEB = EDGE_PER_SUB // EBATCH
NPADN = 10240
ROWS_PER_SUB = NPADN // NS
TROWS = 1024
TROWS_PER_SUB = TROWS // NS

NPAD = 10240
GROWS_PER_W = NPAD * 5 // (NC * NS)
GBATCH = 80
NGB = GROWS_PER_W // GBATCH

TN = 2000
RN = N_NODES // TN
NRES_P = 50176
TK = 896
RK = NRES_P // TK


def _sc_mesh():
    return plsc.VectorSubcoreMesh(
        core_axis_name="c", subcore_axis_name="s", num_cores=NC, num_subcores=NS
    )


_SC_PARAMS = pltpu.CompilerParams(use_tc_tiling_on_sc=False)


NHALF = NPADN // 2
NROWS_ACC = NHALF + 8
STRIPE = NHALF // NS


def _sc_edge_aggregate(etypes, inv, dst, src, ta, tb, zf, zd):
    kfn = pl.kernel(
        _edge_agg_body,
        out_type=(
            jax.ShapeDtypeStruct((NC, NPADN, HHALF), jnp.float32),
            jax.ShapeDtypeStruct((NPADN, 16), jnp.float32),
            jax.ShapeDtypeStruct((NPADN, 16), jnp.float32),
        ),
        mesh=_sc_mesh(),
        scratch_types=[
            pltpu.VMEM_SHARED((NROWS_ACC, HHALF), jnp.float32),
            pltpu.VMEM_SHARED((NPADN, 16), jnp.float32),
            pltpu.VMEM((EBATCH,), jnp.int32),
            pltpu.VMEM((EBATCH,), jnp.int32),
            pltpu.VMEM((EBATCH,), jnp.int32),
            pltpu.VMEM((EBATCH,), jnp.int32),
            pltpu.VMEM((EBATCH,), jnp.int32),
            pltpu.VMEM((EBATCH,), jnp.int32),
            pltpu.VMEM((EBATCH, HHALF), jnp.float32),
            pltpu.VMEM((EBATCH, 16), jnp.float32),
            pltpu.VMEM((STRIPE, HHALF), jnp.float32),
            pltpu.VMEM((ROWS_PER_SUB, 16), jnp.float32),
            pltpu.SemaphoreType.DMA,
        ],
        compiler_params=_SC_PARAMS,
    )
    return kfn(etypes, inv, dst, src, ta, tb, zf, zd)


def _edge_agg_body(et_hbm, iv_hbm, dst_hbm, src_hbm, ta_hbm, tb_hbm,
                   zf_hbm, zd_hbm,
                   featp_hbm, indeg_hbm, outdeg_hbm,
                   fa_sp, dg_sp,
                   et_v, iv_v, code_v, dst_v, sdst_v, src_v, rows_v, ones_v,
                   zbuf_v, zdbuf_v, sem):
    c = lax.axis_index("c")
    s = lax.axis_index("s")
    r0 = s * ROWS_PER_SUB

    for i in range(EBATCH):
        ones_v[i, :] = jnp.ones((16,), jnp.float32)

    pltpu.sync_copy(zd_hbm, zdbuf_v)
    pltpu.sync_copy(zdbuf_v, dg_sp.at[pl.ds(r0, ROWS_PER_SUB)])

    for p in range(2):
        pltpu.sync_copy(zf_hbm, zbuf_v)
        pltpu.sync_copy(zbuf_v, fa_sp.at[pl.ds(s * STRIPE, STRIPE)])

        @pl.when(s == 0)
        def _():
            pltpu.sync_copy(zf_hbm.at[pl.ds(0, 8)], fa_sp.at[pl.ds(NHALF, 8)])

        plsc.subcore_barrier()

        def body(b, carry, p=p):
            base = s * EDGE_PER_SUB + b * EBATCH
            pltpu.sync_copy(et_hbm.at[pl.ds(base, EBATCH)], et_v)
            pltpu.sync_copy(iv_hbm.at[pl.ds(base, EBATCH)], iv_v)
            pltpu.sync_copy(dst_hbm.at[pl.ds(base, EBATCH)], dst_v)
            for i in range(EBATCH // 16):
                sl = pl.ds(i * 16, 16)
                code_v[sl] = et_v[sl] + NREL * iv_v[sl]
                t = dst_v[sl] - (p * NHALF)
                ok = (t >= 0) & (t < NHALF)
                sdst_v[sl] = jnp.where(ok, t, NHALF)

            @pl.when(c == 0)
            def _():
                pltpu.async_copy(ta_hbm.at[code_v], rows_v, sem).wait()

            @pl.when(c == 1)
            def _():
                pltpu.async_copy(tb_hbm.at[code_v], rows_v, sem).wait()

            pltpu.sync_copy(rows_v, fa_sp.at[sdst_v], add=True)

            if p == 0:
                pltpu.sync_copy(src_hbm.at[pl.ds(base, EBATCH)], src_v)

                @pl.when(c == 0)
                def _():
                    pltpu.sync_copy(ones_v, dg_sp.at[dst_v], add=True)

                @pl.when(c == 1)
                def _():
                    pltpu.sync_copy(ones_v, dg_sp.at[src_v], add=True)

            return carry

        lax.fori_loop(0, NEB, body, 0)
        plsc.subcore_barrier()

        pltpu.sync_copy(fa_sp.at[pl.ds(s * STRIPE, STRIPE)], zbuf_v)
        pltpu.sync_copy(zbuf_v,
                        featp_hbm.at[c, pl.ds(p * NHALF + s * STRIPE, STRIPE)])

    pltpu.sync_copy(dg_sp.at[pl.ds(r0, ROWS_PER_SUB)], zdbuf_v)

    @pl.when(c == 0)
    def _():
        pltpu.sync_copy(zdbuf_v, indeg_hbm.at[pl.ds(r0, ROWS_PER_SUB)])

    @pl.when(c == 1)
    def _():
        pltpu.sync_copy(zdbuf_v, outdeg_hbm.at[pl.ds(r0, ROWS_PER_SUB)])


def _sc_gather_rows(res, idxf):
    kfn = pl.kernel(
        _gather_body,
        out_type=jax.ShapeDtypeStruct((NPAD * 5, HID), jnp.float32),
        mesh=_sc_mesh(),
        scratch_types=[
            pltpu.VMEM((GBATCH,), jnp.int32),
            pltpu.VMEM((GBATCH, HID), jnp.float32),
            pltpu.SemaphoreType.DMA,
        ],
        compiler_params=_SC_PARAMS,
    )
    return kfn(res, idxf)


def _gather_body(res_hbm, idx_hbm, out_hbm, idx_v, rows_v, sem):
    wid = lax.axis_index("s") * NC + lax.axis_index("c")

    def body(b, carry):
        base = wid * GROWS_PER_W + b * GBATCH
        pltpu.sync_copy(idx_hbm.at[pl.ds(base, GBATCH)], idx_v)
        pltpu.async_copy(res_hbm.at[idx_v], rows_v, sem).wait()
        pltpu.sync_copy(rows_v, out_hbm.at[pl.ds(base, GBATCH)])
        return carry

    lax.fori_loop(0, NGB, body, 0)


def _knn_body(x_ref, z_ref, idx_ref, w_ref, tv_s, ti_s):
    j = pl.program_id(1)
    x = x_ref[...]
    z = z_ref[...]
    xden = jnp.maximum(jnp.sqrt(jnp.sum(x * x, axis=1, keepdims=True)), 1e-12)
    xb = (x / xden).astype(jnp.bfloat16).astype(jnp.float32)
    zden = jnp.maximum(jnp.sqrt(jnp.sum(z * z, axis=1, keepdims=True)), 1e-12)
    zb = (z / zden).astype(jnp.bfloat16).astype(jnp.float32)
    s = lax.dot_general(xb, zb, (((1,), (1,)), ((), ())),
                        precision=lax.Precision.HIGHEST,
                        preferred_element_type=jnp.float32)
    gcol = j * TK + lax.broadcasted_iota(jnp.int32, (TN, TK), 1)
    s = jnp.where(gcol < NRES, s, -jnp.inf)

    @pl.when(j == 0)
    def _():
        tv_s[...] = jnp.full((TN, 5), -jnp.inf, jnp.float32)
        ti_s[...] = jnp.zeros((TN, 5), jnp.int32)

    old_v = tv_s[...]
    old_i = ti_s[...]
    aug = jnp.concatenate([old_v, s], axis=1)
    aw = 5 + TK
    iota = lax.broadcasted_iota(jnp.int32, (TN, aw), 1)
    iota5 = lax.broadcasted_iota(jnp.int32, (TN, 5), 1)
    nv, ni = [], []
    for _ in range(5):
        m = jnp.max(aug, axis=1, keepdims=True)
        cand = jnp.where(aug == m, iota, aw)
        am = jnp.min(cand, axis=1, keepdims=True)
        old_at = jnp.sum(jnp.where(iota5 == am, old_i, 0), axis=1,
                         keepdims=True)
        gidx = jnp.where(am < 5, old_at, j * TK + (am - 5))
        nv.append(m)
        ni.append(gidx)
        aug = jnp.where(iota == am, -jnp.inf, aug)
    tv = jnp.concatenate(nv, axis=1)
    ti = jnp.concatenate(ni, axis=1)
    tv_s[...] = tv
    ti_s[...] = ti

    @pl.when(j == RK - 1)
    def _():
        tsc = tv / 0.2
        e = jnp.exp(tsc - jnp.max(tsc, axis=1, keepdims=True))
        w_ref[...] = e / jnp.sum(e, axis=1, keepdims=True)
        idx_ref[...] = ti


def _tc_knn(ent, res_p):
    return pl.pallas_call(
        _knn_body,
        grid=(RN, RK),
        in_specs=[
            pl.BlockSpec((TN, HID), lambda i, j: (i, 0)),
            pl.BlockSpec((TK, HID), lambda i, j: (j, 0)),
        ],
        out_specs=[
            pl.BlockSpec((TN, 5), lambda i, j: (i, 0)),
            pl.BlockSpec((TN, 5), lambda i, j: (i, 0)),
        ],
        out_shape=[
            jax.ShapeDtypeStruct((N_NODES, 5), jnp.int32),
            jax.ShapeDtypeStruct((N_NODES, 5), jnp.float32),
        ],
        scratch_shapes=[
            pltpu.VMEM((TN, 5), jnp.float32),
            pltpu.VMEM((TN, 5), jnp.int32),
        ],
    )(ent, res_p)


def _final_body(f_ref, id_ref, od_ref, g_ref, w_ref,
                w1_ref, b1_ref, w2_ref, b2_ref, o_ref):
    ind = id_ref[...][:, 0:1]
    outd = od_ref[...][:, 0:1]
    feat = f_ref[...] / jnp.maximum(ind, 1.0)
    w = w_ref[...]
    g = g_ref[...]
    ze = w[:, 0:1] * g[:, 0:HID]
    for k in range(1, 5):
        ze = ze + w[:, k:k + 1] * g[:, k * HID:(k + 1) * HID]
    feat = jnp.where((ind + outd) == 0.0, ze, feat)
    fb = feat.astype(jnp.bfloat16).astype(jnp.float32)
    w1b = w1_ref[...].astype(jnp.bfloat16).astype(jnp.float32)
    h = lax.dot_general(fb, w1b, (((1,), (1,)), ((), ())),
                        precision=lax.Precision.HIGHEST,
                        preferred_element_type=jnp.float32) + b1_ref[...]
    h = jnp.maximum(h, 0.0)
    hb = h.astype(jnp.bfloat16).astype(jnp.float32)
    w2b = w2_ref[...].astype(jnp.bfloat16).astype(jnp.float32)
    o_ref[...] = lax.dot_general(hb, w2b, (((1,), (1,)), ((), ())),
                                 precision=lax.Precision.HIGHEST,
                                 preferred_element_type=jnp.float32) + b2_ref[...]


def _tc_final(feat_sum, indeg, outdeg, g5, w8, W1, b1, W2, b2):
    return pl.pallas_call(
        _final_body,
        grid=(RN,),
        in_specs=[
            pl.BlockSpec((TN, HID), lambda i: (i, 0)),
            pl.BlockSpec((TN, 16), lambda i: (i, 0)),
            pl.BlockSpec((TN, 16), lambda i: (i, 0)),
            pl.BlockSpec((TN, 5 * HID), lambda i: (i, 0)),
            pl.BlockSpec((TN, 8), lambda i: (i, 0)),
            pl.BlockSpec((HID, HID), lambda i: (0, 0)),
            pl.BlockSpec((1, HID), lambda i: (0, 0)),
            pl.BlockSpec((HID, HID), lambda i: (0, 0)),
            pl.BlockSpec((1, HID), lambda i: (0, 0)),
        ],
        out_specs=pl.BlockSpec((TN, HID), lambda i: (i, 0)),
        out_shape=jax.ShapeDtypeStruct((N_NODES, HID), jnp.float32),
    )(feat_sum, indeg, outdeg, g5, w8, W1, b1, W2, b2)


def kernel(edge_index, etypes, inv, ent_type_feat_emb,
           rel_head_emb, rel_tail_emb, res_ent_emb, W1, b1, W2, b2):
    ei = edge_index.astype(jnp.int32)
    et = etypes.astype(jnp.int32)
    iv = inv.astype(jnp.int32)

    t = jnp.concatenate([rel_tail_emb, rel_head_emb], axis=0)
    t2 = jnp.pad(t, ((0, TROWS - 2 * NREL), (0, 0)))
    ta = t2[:, :HHALF]
    tb = t2[:, HHALF:]
    zf = jnp.zeros((STRIPE, HHALF), jnp.float32)
    zd = jnp.zeros((ROWS_PER_SUB, 16), jnp.float32)

    featp, indeg, outdeg = _sc_edge_aggregate(et, iv, ei[1], ei[0],
                                              ta, tb, zf, zd)
    feat_sum = featp[:, :N_NODES].transpose(1, 0, 2).reshape(N_NODES, HID)
    indeg = indeg[:N_NODES]
    outdeg = outdeg[:N_NODES]

    res_p = jnp.pad(res_ent_emb, ((0, NRES_P - NRES), (0, 0)))
    topk_idx, w = _tc_knn(ent_type_feat_emb, res_p)

    idxf = jnp.pad(topk_idx, ((0, NPAD - N_NODES), (0, 0))).reshape(-1)
    gath = _sc_gather_rows(res_ent_emb, idxf)
    g5 = gath[: N_NODES * 5].reshape(N_NODES, 5 * HID)
    w8 = jnp.pad(w, ((0, 0), (0, 3)))

    return _tc_final(feat_sum, indeg, outdeg, g5, w8,
                     W1, b1.reshape(1, HID), W2, b2.reshape(1, HID))


# speedup vs baseline: 2.3678x; 1.3416x over previous
import functools

import jax
import jax.numpy as jnp
from jax import lax
from jax.experimental import pallas as pl
from jax.experimental.pallas import tpu as pltpu
from jax.experimental.pallas import tpu_sc as plsc

N_NODES = 10000
N_EDGES = 160000
HID = 256
NREL = 500
NRES = 50000

NC = 2
NS = 16
HHALF = HID // 2

EDGE_PER_SUB = N_EDGES // NS
EBATCH = 80
NEB = EDGE_PER_SUB // EBATCH
NPADN = 10240
ROWS_PER_SUB = NPADN // NS
TROWS = 1024
TROWS_PER_SUB = TROWS // NS

NPAD = 10240
GROWS_PER_W = NPAD * 5 // (NC * NS)
GBATCH = 80
NGB = GROWS_PER_W // GBATCH

TN = 2000
RN = N_NODES // TN
NRES_P = 50176
TK = 896
RK = NRES_P // TK


def _sc_mesh():
    return plsc.VectorSubcoreMesh(
        core_axis_name="c", subcore_axis_name="s", num_cores=NC, num_subcores=NS
    )


_SC_PARAMS = pltpu.CompilerParams(use_tc_tiling_on_sc=False)


NHALF = NPADN // 2
NROWS_ACC = NHALF + 8
STRIPE = NHALF // NS


def _sc_edge_aggregate(etypes, inv, dst, src, ta, tb, zf, zd):
    kfn = pl.kernel(
        _edge_agg_body,
        out_type=(
            jax.ShapeDtypeStruct((NC, NPADN, HHALF), jnp.float32),
            jax.ShapeDtypeStruct((NPADN, 16), jnp.float32),
            jax.ShapeDtypeStruct((NPADN, 16), jnp.float32),
        ),
        mesh=_sc_mesh(),
        scratch_types=[
            pltpu.VMEM_SHARED((NROWS_ACC, HHALF), jnp.float32),
            pltpu.VMEM_SHARED((NPADN, 16), jnp.float32),
            pltpu.VMEM((EBATCH,), jnp.int32),
            pltpu.VMEM((EBATCH,), jnp.int32),
            pltpu.VMEM((EBATCH,), jnp.int32),
            pltpu.VMEM((EBATCH,), jnp.int32),
            pltpu.VMEM((EBATCH,), jnp.int32),
            pltpu.VMEM((EBATCH,), jnp.int32),
            pltpu.VMEM((EBATCH, HHALF), jnp.float32),
            pltpu.VMEM((EBATCH, 16), jnp.float32),
            pltpu.VMEM((STRIPE, HHALF), jnp.float32),
            pltpu.VMEM((ROWS_PER_SUB, 16), jnp.float32),
            pltpu.SemaphoreType.DMA,
        ],
        compiler_params=_SC_PARAMS,
    )
    return kfn(etypes, inv, dst, src, ta, tb, zf, zd)


def _edge_agg_body(et_hbm, iv_hbm, dst_hbm, src_hbm, ta_hbm, tb_hbm,
                   zf_hbm, zd_hbm,
                   featp_hbm, indeg_hbm, outdeg_hbm,
                   fa_sp, dg_sp,
                   et_v, iv_v, code_v, dst_v, sdst_v, src_v, rows_v, ones_v,
                   zbuf_v, zdbuf_v, sem):
    c = lax.axis_index("c")
    s = lax.axis_index("s")
    r0 = s * ROWS_PER_SUB

    for i in range(EBATCH):
        ones_v[i, :] = jnp.ones((16,), jnp.float32)

    pltpu.sync_copy(zd_hbm, zdbuf_v)
    pltpu.sync_copy(zdbuf_v, dg_sp.at[pl.ds(r0, ROWS_PER_SUB)])

    for p in range(2):
        pltpu.sync_copy(zf_hbm, zbuf_v)
        pltpu.sync_copy(zbuf_v, fa_sp.at[pl.ds(s * STRIPE, STRIPE)])

        @pl.when(s == 0)
        def _():
            pltpu.sync_copy(zf_hbm.at[pl.ds(0, 8)], fa_sp.at[pl.ds(NHALF, 8)])

        plsc.subcore_barrier()

        def body(b, carry, p=p):
            base = s * EDGE_PER_SUB + b * EBATCH
            pltpu.sync_copy(et_hbm.at[pl.ds(base, EBATCH)], et_v)
            pltpu.sync_copy(iv_hbm.at[pl.ds(base, EBATCH)], iv_v)
            pltpu.sync_copy(dst_hbm.at[pl.ds(base, EBATCH)], dst_v)
            for i in range(EBATCH // 16):
                sl = pl.ds(i * 16, 16)
                code_v[sl] = et_v[sl] + NREL * iv_v[sl]
                t = dst_v[sl] - (p * NHALF)
                ok = (t >= 0) & (t < NHALF)
                sdst_v[sl] = jnp.where(ok, t, NHALF)

            @pl.when(c == 0)
            def _():
                pltpu.async_copy(ta_hbm.at[code_v], rows_v, sem).wait()

            @pl.when(c == 1)
            def _():
                pltpu.async_copy(tb_hbm.at[code_v], rows_v, sem).wait()

            pltpu.sync_copy(rows_v, fa_sp.at[sdst_v], add=True)

            if p == 0:
                pltpu.sync_copy(src_hbm.at[pl.ds(base, EBATCH)], src_v)

                @pl.when(c == 0)
                def _():
                    pltpu.sync_copy(ones_v, dg_sp.at[dst_v], add=True)

                @pl.when(c == 1)
                def _():
                    pltpu.sync_copy(ones_v, dg_sp.at[src_v], add=True)

            return carry

        lax.fori_loop(0, NEB, body, 0)
        plsc.subcore_barrier()

        pltpu.sync_copy(fa_sp.at[pl.ds(s * STRIPE, STRIPE)], zbuf_v)
        pltpu.sync_copy(zbuf_v,
                        featp_hbm.at[c, pl.ds(p * NHALF + s * STRIPE, STRIPE)])

    pltpu.sync_copy(dg_sp.at[pl.ds(r0, ROWS_PER_SUB)], zdbuf_v)

    @pl.when(c == 0)
    def _():
        pltpu.sync_copy(zdbuf_v, indeg_hbm.at[pl.ds(r0, ROWS_PER_SUB)])

    @pl.when(c == 1)
    def _():
        pltpu.sync_copy(zdbuf_v, outdeg_hbm.at[pl.ds(r0, ROWS_PER_SUB)])


def _sc_gather_rows(res, idxf):
    kfn = pl.kernel(
        _gather_body,
        out_type=jax.ShapeDtypeStruct((NPAD * 5, HID), jnp.float32),
        mesh=_sc_mesh(),
        scratch_types=[
            pltpu.VMEM((GBATCH,), jnp.int32),
            pltpu.VMEM((GBATCH, HID), jnp.float32),
            pltpu.SemaphoreType.DMA,
        ],
        compiler_params=_SC_PARAMS,
    )
    return kfn(res, idxf)


def _gather_body(res_hbm, idx_hbm, out_hbm, idx_v, rows_v, sem):
    wid = lax.axis_index("s") * NC + lax.axis_index("c")

    def body(b, carry):
        base = wid * GROWS_PER_W + b * GBATCH
        pltpu.sync_copy(idx_hbm.at[pl.ds(base, GBATCH)], idx_v)
        pltpu.async_copy(res_hbm.at[idx_v], rows_v, sem).wait()
        pltpu.sync_copy(rows_v, out_hbm.at[pl.ds(base, GBATCH)])
        return carry

    lax.fori_loop(0, NGB, body, 0)


def _knn_body(x_ref, z_ref, idx_ref, w_ref, tv_s, ti_s):
    j = pl.program_id(1)
    x = x_ref[...]
    z = z_ref[...]
    xden = jnp.maximum(jnp.sqrt(jnp.sum(x * x, axis=1, keepdims=True)), 1e-12)
    xb = (x / xden).astype(jnp.bfloat16)
    zden = jnp.maximum(jnp.sqrt(jnp.sum(z * z, axis=1, keepdims=True)), 1e-12)
    zb = (z / zden).astype(jnp.bfloat16)
    s = lax.dot_general(xb, zb, (((1,), (1,)), ((), ())),
                        preferred_element_type=jnp.float32)
    gcol = j * TK + lax.broadcasted_iota(jnp.int32, (TN, TK), 1)
    s = jnp.where(gcol < NRES, s, -jnp.inf)

    @pl.when(j == 0)
    def _():
        tv_s[...] = jnp.full((TN, 5), -jnp.inf, jnp.float32)
        ti_s[...] = jnp.zeros((TN, 5), jnp.int32)

    old_v = tv_s[...]
    old_i = ti_s[...]
    aug = jnp.concatenate([old_v, s], axis=1)
    aw = 5 + TK
    iota = lax.broadcasted_iota(jnp.int32, (TN, aw), 1)
    iota5 = lax.broadcasted_iota(jnp.int32, (TN, 5), 1)
    nv, ni = [], []
    for _ in range(5):
        m = jnp.max(aug, axis=1, keepdims=True)
        cand = jnp.where(aug == m, iota, aw)
        am = jnp.min(cand, axis=1, keepdims=True)
        old_at = jnp.sum(jnp.where(iota5 == am, old_i, 0), axis=1,
                         keepdims=True)
        gidx = jnp.where(am < 5, old_at, j * TK + (am - 5))
        nv.append(m)
        ni.append(gidx)
        aug = jnp.where(iota == am, -jnp.inf, aug)
    tv = jnp.concatenate(nv, axis=1)
    ti = jnp.concatenate(ni, axis=1)
    tv_s[...] = tv
    ti_s[...] = ti

    @pl.when(j == RK - 1)
    def _():
        tsc = tv / 0.2
        e = jnp.exp(tsc - jnp.max(tsc, axis=1, keepdims=True))
        w_ref[...] = e / jnp.sum(e, axis=1, keepdims=True)
        idx_ref[...] = ti


def _tc_knn(ent, res_p):
    return pl.pallas_call(
        _knn_body,
        grid=(RN, RK),
        in_specs=[
            pl.BlockSpec((TN, HID), lambda i, j: (i, 0)),
            pl.BlockSpec((TK, HID), lambda i, j: (j, 0)),
        ],
        out_specs=[
            pl.BlockSpec((TN, 5), lambda i, j: (i, 0)),
            pl.BlockSpec((TN, 5), lambda i, j: (i, 0)),
        ],
        out_shape=[
            jax.ShapeDtypeStruct((N_NODES, 5), jnp.int32),
            jax.ShapeDtypeStruct((N_NODES, 5), jnp.float32),
        ],
        scratch_shapes=[
            pltpu.VMEM((TN, 5), jnp.float32),
            pltpu.VMEM((TN, 5), jnp.int32),
        ],
    )(ent, res_p)


def _final_body(fa_ref, fb_ref, id_ref, od_ref, g_ref, w_ref,
                w1_ref, b1_ref, w2_ref, b2_ref, o_ref):
    ind = id_ref[...][:, 0:1]
    outd = od_ref[...][:, 0:1]
    fsum = jnp.concatenate([fa_ref[...], fb_ref[...]], axis=1)
    feat = fsum / jnp.maximum(ind, 1.0)
    w = w_ref[...]
    g = g_ref[...]
    ze = w[:, 0:1] * g[:, 0:HID]
    for k in range(1, 5):
        ze = ze + w[:, k:k + 1] * g[:, k * HID:(k + 1) * HID]
    feat = jnp.where((ind + outd) == 0.0, ze, feat)
    fb = feat.astype(jnp.bfloat16)
    w1b = w1_ref[...].astype(jnp.bfloat16)
    h = lax.dot_general(fb, w1b, (((1,), (1,)), ((), ())),
                        preferred_element_type=jnp.float32) + b1_ref[...]
    h = jnp.maximum(h, 0.0)
    hb = h.astype(jnp.bfloat16)
    w2b = w2_ref[...].astype(jnp.bfloat16)
    o_ref[...] = lax.dot_general(hb, w2b, (((1,), (1,)), ((), ())),
                                 preferred_element_type=jnp.float32) + b2_ref[...]


def _tc_final(fa, fb, indeg, outdeg, g5, w8, W1, b1, W2, b2):
    return pl.pallas_call(
        _final_body,
        grid=(RN,),
        in_specs=[
            pl.BlockSpec((TN, HHALF), lambda i: (i, 0)),
            pl.BlockSpec((TN, HHALF), lambda i: (i, 0)),
            pl.BlockSpec((TN, 16), lambda i: (i, 0)),
            pl.BlockSpec((TN, 16), lambda i: (i, 0)),
            pl.BlockSpec((TN, 5 * HID), lambda i: (i, 0)),
            pl.BlockSpec((TN, 8), lambda i: (i, 0)),
            pl.BlockSpec((HID, HID), lambda i: (0, 0)),
            pl.BlockSpec((1, HID), lambda i: (0, 0)),
            pl.BlockSpec((HID, HID), lambda i: (0, 0)),
            pl.BlockSpec((1, HID), lambda i: (0, 0)),
        ],
        out_specs=pl.BlockSpec((TN, HID), lambda i: (i, 0)),
        out_shape=jax.ShapeDtypeStruct((N_NODES, HID), jnp.float32),
    )(fa, fb, indeg, outdeg, g5, w8, W1, b1, W2, b2)


def kernel(edge_index, etypes, inv, ent_type_feat_emb,
           rel_head_emb, rel_tail_emb, res_ent_emb, W1, b1, W2, b2):
    ei = edge_index.astype(jnp.int32)
    et = etypes.astype(jnp.int32)
    iv = inv.astype(jnp.int32)

    t = jnp.concatenate([rel_tail_emb, rel_head_emb], axis=0)
    t2 = jnp.pad(t, ((0, TROWS - 2 * NREL), (0, 0)))
    ta = t2[:, :HHALF]
    tb = t2[:, HHALF:]
    zf = jnp.zeros((STRIPE, HHALF), jnp.float32)
    zd = jnp.zeros((ROWS_PER_SUB, 16), jnp.float32)

    featp, indeg, outdeg = _sc_edge_aggregate(et, iv, ei[1], ei[0],
                                              ta, tb, zf, zd)

    topk_idx, w = _tc_knn(ent_type_feat_emb, res_ent_emb)

    idxf = jnp.pad(topk_idx, ((0, NPAD - N_NODES), (0, 0))).reshape(-1)
    gath = _sc_gather_rows(res_ent_emb, idxf)
    g5 = gath.reshape(NPAD, 5 * HID)
    w8 = jnp.pad(w, ((0, 0), (0, 3)))

    return _tc_final(featp[0], featp[1], indeg, outdeg, g5, w8,
                     W1, b1.reshape(1, HID), W2, b2.reshape(1, HID))


# speedup vs baseline: 5.8737x; 2.4807x over previous
import functools

import jax
import jax.numpy as jnp
from jax import lax
from jax.experimental import pallas as pl
from jax.experimental.pallas import tpu as pltpu
from jax.experimental.pallas import tpu_sc as plsc

N_NODES = 10000
N_EDGES = 160000
HID = 256
NREL = 500
NRES = 50000

NC = 2
NS = 16
HHALF = HID // 2

EDGE_PER_SUB = N_EDGES // NS
EBATCH = 80
NEB = EDGE_PER_SUB // EBATCH
NPADN = 10240
ROWS_PER_SUB = NPADN // NS
TROWS = 1024
TROWS_PER_SUB = TROWS // NS

NPAD = 10240
GROWS_PER_W = NPAD * 5 // (NC * NS)
GBATCH = 80
NGB = GROWS_PER_W // GBATCH

TN = 2000
RN = N_NODES // TN
NRES_P = 50176
TK = 896
RK = NRES_P // TK


def _sc_mesh():
    return plsc.VectorSubcoreMesh(
        core_axis_name="c", subcore_axis_name="s", num_cores=NC, num_subcores=NS
    )


_SC_PARAMS = pltpu.CompilerParams(use_tc_tiling_on_sc=False)


NHALF = NPADN // 2
NROWS_ACC = NHALF + 8
STRIPE = NHALF // NS


def _sc_edge_aggregate(etypes, inv, dst, src, ta, tb, zf, zd):
    kfn = pl.kernel(
        _edge_agg_body,
        out_type=(
            jax.ShapeDtypeStruct((NC, NPADN, HHALF), jnp.float32),
            jax.ShapeDtypeStruct((NPADN, 16), jnp.float32),
            jax.ShapeDtypeStruct((NPADN, 16), jnp.float32),
        ),
        mesh=_sc_mesh(),
        scratch_types=[
            pltpu.VMEM_SHARED((NROWS_ACC, HHALF), jnp.float32),
            pltpu.VMEM_SHARED((NPADN, 16), jnp.float32),
            pltpu.VMEM((EBATCH,), jnp.int32),
            pltpu.VMEM((EBATCH,), jnp.int32),
            pltpu.VMEM((EBATCH,), jnp.int32),
            pltpu.VMEM((EBATCH,), jnp.int32),
            pltpu.VMEM((EBATCH,), jnp.int32),
            pltpu.VMEM((EBATCH,), jnp.int32),
            pltpu.VMEM((EBATCH, HHALF), jnp.float32),
            pltpu.VMEM((EBATCH, 16), jnp.float32),
            pltpu.VMEM((STRIPE, HHALF), jnp.float32),
            pltpu.VMEM((ROWS_PER_SUB, 16), jnp.float32),
            pltpu.SemaphoreType.DMA,
        ],
        compiler_params=_SC_PARAMS,
    )
    return kfn(etypes, inv, dst, src, ta, tb, zf, zd)


def _edge_agg_body(et_hbm, iv_hbm, dst_hbm, src_hbm, ta_hbm, tb_hbm,
                   zf_hbm, zd_hbm,
                   featp_hbm, indeg_hbm, outdeg_hbm,
                   fa_sp, dg_sp,
                   et_v, iv_v, code_v, dst_v, sdst_v, src_v, rows_v, ones_v,
                   zbuf_v, zdbuf_v, sem):
    c = lax.axis_index("c")
    s = lax.axis_index("s")
    r0 = s * ROWS_PER_SUB

    for i in range(EBATCH):
        ones_v[i, :] = jnp.ones((16,), jnp.float32)

    pltpu.sync_copy(zd_hbm, zdbuf_v)
    pltpu.sync_copy(zdbuf_v, dg_sp.at[pl.ds(r0, ROWS_PER_SUB)])

    for p in range(2):
        pltpu.sync_copy(zf_hbm, zbuf_v)
        pltpu.sync_copy(zbuf_v, fa_sp.at[pl.ds(s * STRIPE, STRIPE)])

        @pl.when(s == 0)
        def _():
            pltpu.sync_copy(zf_hbm.at[pl.ds(0, 8)], fa_sp.at[pl.ds(NHALF, 8)])

        plsc.subcore_barrier()

        def body(b, carry, p=p):
            base = s * EDGE_PER_SUB + b * EBATCH
            pltpu.sync_copy(et_hbm.at[pl.ds(base, EBATCH)], et_v)
            pltpu.sync_copy(iv_hbm.at[pl.ds(base, EBATCH)], iv_v)
            pltpu.sync_copy(dst_hbm.at[pl.ds(base, EBATCH)], dst_v)
            for i in range(EBATCH // 16):
                sl = pl.ds(i * 16, 16)
                code_v[sl] = et_v[sl] + NREL * iv_v[sl]
                t = dst_v[sl] - (p * NHALF)
                ok = (t >= 0) & (t < NHALF)
                sdst_v[sl] = jnp.where(ok, t, NHALF)

            @pl.when(c == 0)
            def _():
                pltpu.async_copy(ta_hbm.at[code_v], rows_v, sem).wait()

            @pl.when(c == 1)
            def _():
                pltpu.async_copy(tb_hbm.at[code_v], rows_v, sem).wait()

            pltpu.sync_copy(rows_v, fa_sp.at[sdst_v], add=True)

            if p == 0:
                pltpu.sync_copy(src_hbm.at[pl.ds(base, EBATCH)], src_v)

                @pl.when(c == 0)
                def _():
                    pltpu.sync_copy(ones_v, dg_sp.at[dst_v], add=True)

                @pl.when(c == 1)
                def _():
                    pltpu.sync_copy(ones_v, dg_sp.at[src_v], add=True)

            return carry

        lax.fori_loop(0, NEB, body, 0)
        plsc.subcore_barrier()

        pltpu.sync_copy(fa_sp.at[pl.ds(s * STRIPE, STRIPE)], zbuf_v)
        pltpu.sync_copy(zbuf_v,
                        featp_hbm.at[c, pl.ds(p * NHALF + s * STRIPE, STRIPE)])

    pltpu.sync_copy(dg_sp.at[pl.ds(r0, ROWS_PER_SUB)], zdbuf_v)

    @pl.when(c == 0)
    def _():
        pltpu.sync_copy(zdbuf_v, indeg_hbm.at[pl.ds(r0, ROWS_PER_SUB)])

    @pl.when(c == 1)
    def _():
        pltpu.sync_copy(zdbuf_v, outdeg_hbm.at[pl.ds(r0, ROWS_PER_SUB)])


def _sc_gather_rows(res, idxf):
    kfn = pl.kernel(
        _gather_body,
        out_type=jax.ShapeDtypeStruct((NPAD * 5, HID), jnp.float32),
        mesh=_sc_mesh(),
        scratch_types=[
            pltpu.VMEM((GBATCH,), jnp.int32),
            pltpu.VMEM((GBATCH, HID), jnp.float32),
            pltpu.SemaphoreType.DMA,
        ],
        compiler_params=_SC_PARAMS,
    )
    return kfn(res, idxf)


def _gather_body(res_hbm, idx_hbm, out_hbm, idx_v, rows_v, sem):
    wid = lax.axis_index("s") * NC + lax.axis_index("c")

    def body(b, carry):
        base = wid * GROWS_PER_W + b * GBATCH
        pltpu.sync_copy(idx_hbm.at[pl.ds(base, GBATCH)], idx_v)
        pltpu.async_copy(res_hbm.at[idx_v], rows_v, sem).wait()
        pltpu.sync_copy(rows_v, out_hbm.at[pl.ds(base, GBATCH)])
        return carry

    lax.fori_loop(0, NGB, body, 0)


P_COMP = 2048
CROWS_PER_W = P_COMP // (NC * NS)
CNB = 4
CNODES_B = CROWS_PER_W // CNB


def _sc_gather_xcomp(ent, comp_idx):
    kfn = pl.kernel(
        _gather_xcomp_body,
        out_type=jax.ShapeDtypeStruct((P_COMP, HID), jnp.float32),
        mesh=_sc_mesh(),
        scratch_types=[
            pltpu.VMEM((CROWS_PER_W,), jnp.int32),
            pltpu.VMEM((CROWS_PER_W, HID), jnp.float32),
            pltpu.SemaphoreType.DMA,
        ],
        compiler_params=_SC_PARAMS,
    )
    return kfn(ent, comp_idx)


def _gather_xcomp_body(ent_hbm, idx_hbm, out_hbm, idx_v, rows_v, sem):
    wid = lax.axis_index("s") * NC + lax.axis_index("c")
    base = wid * CROWS_PER_W
    pltpu.sync_copy(idx_hbm.at[pl.ds(base, CROWS_PER_W)], idx_v)
    pltpu.async_copy(ent_hbm.at[idx_v], rows_v, sem).wait()
    pltpu.sync_copy(rows_v, out_hbm.at[pl.ds(base, CROWS_PER_W)])


def _sc_gather_scatter_comp(res, tk5, scat5, w16c, nid):
    kfn = pl.kernel(
        _gs_comp_body,
        out_type=(
            jax.ShapeDtypeStruct((NPAD * 5, HID), jnp.float32),
            jax.ShapeDtypeStruct((NPAD, 16), jnp.float32),
        ),
        mesh=_sc_mesh(),
        scratch_types=[
            pltpu.VMEM((CNODES_B * 5,), jnp.int32),
            pltpu.VMEM((CNODES_B * 5,), jnp.int32),
            pltpu.VMEM((CNODES_B,), jnp.int32),
            pltpu.VMEM((CNODES_B * 5, HID), jnp.float32),
            pltpu.VMEM((CNODES_B, 16), jnp.float32),
            pltpu.SemaphoreType.DMA,
            pltpu.SemaphoreType.DMA,
            pltpu.SemaphoreType.DMA,
        ],
        compiler_params=_SC_PARAMS,
    )
    return kfn(res, tk5, scat5, w16c, nid)


def _gs_comp_body(res_hbm, tk5_hbm, scat5_hbm, w16c_hbm, nid_hbm,
                  g5f_hbm, w16d_hbm,
                  tk_v, sc_v, nid_v, rows_v, w_v, sem, sem2, sem3):
    wid = lax.axis_index("s") * NC + lax.axis_index("c")

    def body(b, carry):
        nb = wid * CROWS_PER_W + b * CNODES_B
        rb = nb * 5
        pltpu.sync_copy(tk5_hbm.at[pl.ds(rb, CNODES_B * 5)], tk_v)
        pltpu.sync_copy(scat5_hbm.at[pl.ds(rb, CNODES_B * 5)], sc_v)
        pltpu.sync_copy(nid_hbm.at[pl.ds(nb, CNODES_B)], nid_v)
        pltpu.sync_copy(w16c_hbm.at[pl.ds(nb, CNODES_B)], w_v)
        pltpu.async_copy(res_hbm.at[tk_v], rows_v, sem).wait()
        pltpu.async_copy(rows_v, g5f_hbm.at[sc_v], sem2).wait()
        pltpu.async_copy(w_v, w16d_hbm.at[nid_v], sem3).wait()
        return carry

    lax.fori_loop(0, CNB, body, 0)


def _knn_body(x_ref, z_ref, idx_ref, w_ref, tv_s, ti_s):
    j = pl.program_id(1)
    x = x_ref[...]
    z = z_ref[...]
    tn = x.shape[0]
    xden = jnp.maximum(jnp.sqrt(jnp.sum(x * x, axis=1, keepdims=True)), 1e-12)
    xb = (x / xden).astype(jnp.bfloat16)
    zden = jnp.maximum(jnp.sqrt(jnp.sum(z * z, axis=1, keepdims=True)), 1e-12)
    zb = (z / zden).astype(jnp.bfloat16)
    s = lax.dot_general(xb, zb, (((1,), (1,)), ((), ())),
                        preferred_element_type=jnp.float32)
    gcol = j * TK + lax.broadcasted_iota(jnp.int32, (tn, TK), 1)
    s = jnp.where(gcol < NRES, s, -jnp.inf)

    @pl.when(j == 0)
    def _():
        tv_s[...] = jnp.full((tn, 5), -jnp.inf, jnp.float32)
        ti_s[...] = jnp.zeros((tn, 5), jnp.int32)

    old_v = tv_s[...]
    old_i = ti_s[...]
    aug = jnp.concatenate([old_v, s], axis=1)
    aw = 5 + TK
    iota = lax.broadcasted_iota(jnp.int32, (tn, aw), 1)
    iota5 = lax.broadcasted_iota(jnp.int32, (tn, 5), 1)
    nv, ni = [], []
    for _ in range(5):
        m = jnp.max(aug, axis=1, keepdims=True)
        cand = jnp.where(aug == m, iota, aw)
        am = jnp.min(cand, axis=1, keepdims=True)
        old_at = jnp.sum(jnp.where(iota5 == am, old_i, 0), axis=1,
                         keepdims=True)
        gidx = jnp.where(am < 5, old_at, j * TK + (am - 5))
        nv.append(m)
        ni.append(gidx)
        aug = jnp.where(iota == am, -jnp.inf, aug)
    tv = jnp.concatenate(nv, axis=1)
    ti = jnp.concatenate(ni, axis=1)
    tv_s[...] = tv
    ti_s[...] = ti

    @pl.when(j == RK - 1)
    def _():
        tsc = tv / 0.2
        e = jnp.exp(tsc - jnp.max(tsc, axis=1, keepdims=True))
        w_ref[...] = e / jnp.sum(e, axis=1, keepdims=True)
        idx_ref[...] = ti


def _tc_knn(x, res, tn):
    n = x.shape[0]
    return pl.pallas_call(
        _knn_body,
        grid=(n // tn, RK),
        in_specs=[
            pl.BlockSpec((tn, HID), lambda i, j: (i, 0)),
            pl.BlockSpec((TK, HID), lambda i, j: (j, 0)),
        ],
        out_specs=[
            pl.BlockSpec((tn, 5), lambda i, j: (i, 0)),
            pl.BlockSpec((tn, 5), lambda i, j: (i, 0)),
        ],
        out_shape=[
            jax.ShapeDtypeStruct((n, 5), jnp.int32),
            jax.ShapeDtypeStruct((n, 5), jnp.float32),
        ],
        scratch_shapes=[
            pltpu.VMEM((tn, 5), jnp.float32),
            pltpu.VMEM((tn, 5), jnp.int32),
        ],
    )(x, res)


def _final_body(fa_ref, fb_ref, id_ref, od_ref, g_ref, w_ref,
                w1_ref, b1_ref, w2_ref, b2_ref, o_ref):
    ind = id_ref[...][:, 0:1]
    outd = od_ref[...][:, 0:1]
    fsum = jnp.concatenate([fa_ref[...], fb_ref[...]], axis=1)
    feat = fsum / jnp.maximum(ind, 1.0)
    w = w_ref[...]
    g = g_ref[...]
    ze = w[:, 0:1] * g[:, 0:HID]
    for k in range(1, 5):
        ze = ze + w[:, k:k + 1] * g[:, k * HID:(k + 1) * HID]
    feat = jnp.where((ind + outd) == 0.0, ze, feat)
    fb = feat.astype(jnp.bfloat16)
    w1b = w1_ref[...].astype(jnp.bfloat16)
    h = lax.dot_general(fb, w1b, (((1,), (1,)), ((), ())),
                        preferred_element_type=jnp.float32) + b1_ref[...]
    h = jnp.maximum(h, 0.0)
    hb = h.astype(jnp.bfloat16)
    w2b = w2_ref[...].astype(jnp.bfloat16)
    o_ref[...] = lax.dot_general(hb, w2b, (((1,), (1,)), ((), ())),
                                 preferred_element_type=jnp.float32) + b2_ref[...]


def _tc_final(fa, fb, indeg, outdeg, g5, w8, W1, b1, W2, b2):
    return pl.pallas_call(
        _final_body,
        grid=(RN,),
        in_specs=[
            pl.BlockSpec((TN, HHALF), lambda i: (i, 0)),
            pl.BlockSpec((TN, HHALF), lambda i: (i, 0)),
            pl.BlockSpec((TN, 16), lambda i: (i, 0)),
            pl.BlockSpec((TN, 16), lambda i: (i, 0)),
            pl.BlockSpec((TN, 5 * HID), lambda i: (i, 0)),
            pl.BlockSpec((TN, 16), lambda i: (i, 0)),
            pl.BlockSpec((HID, HID), lambda i: (0, 0)),
            pl.BlockSpec((1, HID), lambda i: (0, 0)),
            pl.BlockSpec((HID, HID), lambda i: (0, 0)),
            pl.BlockSpec((1, HID), lambda i: (0, 0)),
        ],
        out_specs=pl.BlockSpec((TN, HID), lambda i: (i, 0)),
        out_shape=jax.ShapeDtypeStruct((N_NODES, HID), jnp.float32),
    )(fa, fb, indeg, outdeg, g5, w8, W1, b1, W2, b2)


def kernel(edge_index, etypes, inv, ent_type_feat_emb,
           rel_head_emb, rel_tail_emb, res_ent_emb, W1, b1, W2, b2):
    ei = edge_index.astype(jnp.int32)
    et = etypes.astype(jnp.int32)
    iv = inv.astype(jnp.int32)

    t = jnp.concatenate([rel_tail_emb, rel_head_emb], axis=0)
    t2 = jnp.pad(t, ((0, TROWS - 2 * NREL), (0, 0)))
    ta = t2[:, :HHALF]
    tb = t2[:, HHALF:]
    zf = jnp.zeros((STRIPE, HHALF), jnp.float32)
    zd = jnp.zeros((ROWS_PER_SUB, 16), jnp.float32)

    featp, indeg, outdeg = _sc_edge_aggregate(et, iv, ei[1], ei[0],
                                              ta, tb, zf, zd)

    ent = ent_type_feat_emb
    res = res_ent_emb
    zmask = (indeg[:N_NODES, 0] + outdeg[:N_NODES, 0]) == 0.0
    z_count = jnp.sum(zmask.astype(jnp.int32))
    pos = jnp.cumsum(zmask.astype(jnp.int32)) - 1
    scat_pos = jnp.where(zmask, pos, P_COMP)
    comp_idx = (jnp.zeros((P_COMP,), jnp.int32)
                .at[scat_pos].set(jnp.arange(N_NODES, dtype=jnp.int32),
                                  mode="drop"))

    def comp_branch(_):
        xc = _sc_gather_xcomp(ent, comp_idx)
        tki, wc = _tc_knn(xc, res, P_COMP)
        tk5 = tki.reshape(-1)
        scat5 = (comp_idx[:, None] * 5
                 + jnp.arange(5, dtype=jnp.int32)[None, :]).reshape(-1)
        w16c = jnp.pad(wc, ((0, 0), (0, 11)))
        g5f, w16d = _sc_gather_scatter_comp(res, tk5, scat5, w16c, comp_idx)
        return g5f.reshape(NPAD, 5 * HID), w16d

    def dense_branch(_):
        tki, w = _tc_knn(ent, res, TN)
        idxf = jnp.pad(tki, ((0, NPAD - N_NODES), (0, 0))).reshape(-1)
        gath = _sc_gather_rows(res, idxf)
        w16 = jnp.pad(w, ((0, NPAD - N_NODES), (0, 11)))
        return gath.reshape(NPAD, 5 * HID), w16

    g5, w16d = lax.cond(z_count <= P_COMP, comp_branch, dense_branch, None)

    return _tc_final(featp[0], featp[1], indeg, outdeg, g5, w16d,
                     W1, b1.reshape(1, HID), W2, b2.reshape(1, HID))


# speedup vs baseline: 6.0375x; 1.0279x over previous
import functools

import jax
import jax.numpy as jnp
from jax import lax
from jax.experimental import pallas as pl
from jax.experimental.pallas import tpu as pltpu
from jax.experimental.pallas import tpu_sc as plsc

N_NODES = 10000
N_EDGES = 160000
HID = 256
NREL = 500
NRES = 50000

NC = 2
NS = 16
HHALF = HID // 2

EDGE_PER_SUB = N_EDGES // NS
EBATCH = 80
NEB = EDGE_PER_SUB // EBATCH
NPADN = 10240
ROWS_PER_SUB = NPADN // NS
TROWS = 1024
TROWS_PER_SUB = TROWS // NS

NPAD = 10240
GROWS_PER_W = NPAD * 5 // (NC * NS)
GBATCH = 80
NGB = GROWS_PER_W // GBATCH

TN = 2000
RN = N_NODES // TN
NRES_P = 50176
TK = 896
RK = NRES_P // TK


def _sc_mesh():
    return plsc.VectorSubcoreMesh(
        core_axis_name="c", subcore_axis_name="s", num_cores=NC, num_subcores=NS
    )


_SC_PARAMS = pltpu.CompilerParams(use_tc_tiling_on_sc=False)


NHALF = NPADN // 2
NROWS_ACC = NHALF + 8
STRIPE = NHALF // NS


def _sc_edge_aggregate(etypes, inv, dst, ta, tb, zf):
    kfn = pl.kernel(
        _edge_agg_body,
        out_type=jax.ShapeDtypeStruct((NC, NPADN, HHALF), jnp.float32),
        mesh=_sc_mesh(),
        scratch_types=[
            pltpu.VMEM_SHARED((NROWS_ACC, HHALF), jnp.float32),
            pltpu.VMEM((EBATCH,), jnp.int32),
            pltpu.VMEM((EBATCH,), jnp.int32),
            pltpu.VMEM((EBATCH,), jnp.int32),
            pltpu.VMEM((EBATCH,), jnp.int32),
            pltpu.VMEM((EBATCH,), jnp.int32),
            pltpu.VMEM((EBATCH, HHALF), jnp.float32),
            pltpu.VMEM((STRIPE, HHALF), jnp.float32),
            pltpu.SemaphoreType.DMA,
        ],
        compiler_params=_SC_PARAMS,
    )
    return kfn(etypes, inv, dst, ta, tb, zf)


def _edge_agg_body(et_hbm, iv_hbm, dst_hbm, ta_hbm, tb_hbm, zf_hbm,
                   featp_hbm,
                   fa_sp,
                   et_v, iv_v, code_v, dst_v, sdst_v, rows_v,
                   zbuf_v, sem):
    c = lax.axis_index("c")
    s = lax.axis_index("s")

    for p in range(2):
        pltpu.sync_copy(zf_hbm, zbuf_v)
        pltpu.sync_copy(zbuf_v, fa_sp.at[pl.ds(s * STRIPE, STRIPE)])

        @pl.when(s == 0)
        def _():
            pltpu.sync_copy(zf_hbm.at[pl.ds(0, 8)], fa_sp.at[pl.ds(NHALF, 8)])

        plsc.subcore_barrier()

        def body(b, carry, p=p):
            base = s * EDGE_PER_SUB + b * EBATCH
            pltpu.sync_copy(et_hbm.at[pl.ds(base, EBATCH)], et_v)
            pltpu.sync_copy(iv_hbm.at[pl.ds(base, EBATCH)], iv_v)
            pltpu.sync_copy(dst_hbm.at[pl.ds(base, EBATCH)], dst_v)
            for i in range(EBATCH // 16):
                sl = pl.ds(i * 16, 16)
                code_v[sl] = et_v[sl] + NREL * iv_v[sl]
                t = dst_v[sl] - (p * NHALF)
                ok = (t >= 0) & (t < NHALF)
                sdst_v[sl] = jnp.where(ok, t, NHALF)

            @pl.when(c == 0)
            def _():
                pltpu.async_copy(ta_hbm.at[code_v], rows_v, sem).wait()

            @pl.when(c == 1)
            def _():
                pltpu.async_copy(tb_hbm.at[code_v], rows_v, sem).wait()

            pltpu.sync_copy(rows_v, fa_sp.at[sdst_v], add=True)
            return carry

        lax.fori_loop(0, NEB, body, 0)
        plsc.subcore_barrier()

        pltpu.sync_copy(fa_sp.at[pl.ds(s * STRIPE, STRIPE)], zbuf_v)
        pltpu.sync_copy(zbuf_v,
                        featp_hbm.at[c, pl.ds(p * NHALF + s * STRIPE, STRIPE)])


def _sc_degrees(dst, src, zd):
    kfn = pl.kernel(
        _degrees_body,
        out_type=(
            jax.ShapeDtypeStruct((NPADN, 16), jnp.float32),
            jax.ShapeDtypeStruct((NPADN, 16), jnp.float32),
        ),
        mesh=_sc_mesh(),
        scratch_types=[
            pltpu.VMEM_SHARED((NPADN, 16), jnp.float32),
            pltpu.VMEM((EBATCH,), jnp.int32),
            pltpu.VMEM((EBATCH, 16), jnp.float32),
            pltpu.VMEM((ROWS_PER_SUB, 16), jnp.float32),
        ],
        compiler_params=_SC_PARAMS,
    )
    return kfn(dst, src, zd)


def _degrees_body(dst_hbm, src_hbm, zd_hbm,
                  indeg_hbm, outdeg_hbm,
                  dg_sp, tgt_v, ones_v, zdbuf_v):
    c = lax.axis_index("c")
    s = lax.axis_index("s")
    r0 = s * ROWS_PER_SUB

    for i in range(EBATCH):
        ones_v[i, :] = jnp.ones((16,), jnp.float32)

    pltpu.sync_copy(zd_hbm, zdbuf_v)
    pltpu.sync_copy(zdbuf_v, dg_sp.at[pl.ds(r0, ROWS_PER_SUB)])
    plsc.subcore_barrier()

    def body(b, carry):
        base = s * EDGE_PER_SUB + b * EBATCH

        @pl.when(c == 0)
        def _():
            pltpu.sync_copy(dst_hbm.at[pl.ds(base, EBATCH)], tgt_v)

        @pl.when(c == 1)
        def _():
            pltpu.sync_copy(src_hbm.at[pl.ds(base, EBATCH)], tgt_v)

        pltpu.sync_copy(ones_v, dg_sp.at[tgt_v], add=True)
        return carry

    lax.fori_loop(0, NEB, body, 0)
    plsc.subcore_barrier()
    pltpu.sync_copy(dg_sp.at[pl.ds(r0, ROWS_PER_SUB)], zdbuf_v)

    @pl.when(c == 0)
    def _():
        pltpu.sync_copy(zdbuf_v, indeg_hbm.at[pl.ds(r0, ROWS_PER_SUB)])

    @pl.when(c == 1)
    def _():
        pltpu.sync_copy(zdbuf_v, outdeg_hbm.at[pl.ds(r0, ROWS_PER_SUB)])


def _sc_gather_rows(res, idxf):
    kfn = pl.kernel(
        _gather_body,
        out_type=jax.ShapeDtypeStruct((NPAD * 5, HID), jnp.float32),
        mesh=_sc_mesh(),
        scratch_types=[
            pltpu.VMEM((GBATCH,), jnp.int32),
            pltpu.VMEM((GBATCH, HID), jnp.float32),
            pltpu.SemaphoreType.DMA,
        ],
        compiler_params=_SC_PARAMS,
    )
    return kfn(res, idxf)


def _gather_body(res_hbm, idx_hbm, out_hbm, idx_v, rows_v, sem):
    wid = lax.axis_index("s") * NC + lax.axis_index("c")

    def body(b, carry):
        base = wid * GROWS_PER_W + b * GBATCH
        pltpu.sync_copy(idx_hbm.at[pl.ds(base, GBATCH)], idx_v)
        pltpu.async_copy(res_hbm.at[idx_v], rows_v, sem).wait()
        pltpu.sync_copy(rows_v, out_hbm.at[pl.ds(base, GBATCH)])
        return carry

    lax.fori_loop(0, NGB, body, 0)


P_COMP = 2048
CROWS_PER_W = P_COMP // (NC * NS)
CNB = 4
CNODES_B = CROWS_PER_W // CNB


def _sc_gather_xcomp(ent, comp_idx):
    kfn = pl.kernel(
        _gather_xcomp_body,
        out_type=jax.ShapeDtypeStruct((P_COMP, HID), jnp.float32),
        mesh=_sc_mesh(),
        scratch_types=[
            pltpu.VMEM((CROWS_PER_W,), jnp.int32),
            pltpu.VMEM((CROWS_PER_W, HID), jnp.float32),
            pltpu.SemaphoreType.DMA,
        ],
        compiler_params=_SC_PARAMS,
    )
    return kfn(ent, comp_idx)


def _gather_xcomp_body(ent_hbm, idx_hbm, out_hbm, idx_v, rows_v, sem):
    wid = lax.axis_index("s") * NC + lax.axis_index("c")
    base = wid * CROWS_PER_W
    pltpu.sync_copy(idx_hbm.at[pl.ds(base, CROWS_PER_W)], idx_v)
    pltpu.async_copy(ent_hbm.at[idx_v], rows_v, sem).wait()
    pltpu.sync_copy(rows_v, out_hbm.at[pl.ds(base, CROWS_PER_W)])


def _sc_gather_scatter_comp(res, tk5, scat5, w16c, nid):
    kfn = pl.kernel(
        _gs_comp_body,
        out_type=(
            jax.ShapeDtypeStruct((NPAD * 5, HID), jnp.float32),
            jax.ShapeDtypeStruct((NPAD, 16), jnp.float32),
        ),
        mesh=_sc_mesh(),
        scratch_types=[
            pltpu.VMEM((CNODES_B * 5,), jnp.int32),
            pltpu.VMEM((CNODES_B * 5,), jnp.int32),
            pltpu.VMEM((CNODES_B,), jnp.int32),
            pltpu.VMEM((CNODES_B * 5, HID), jnp.float32),
            pltpu.VMEM((CNODES_B, 16), jnp.float32),
            pltpu.SemaphoreType.DMA,
            pltpu.SemaphoreType.DMA,
            pltpu.SemaphoreType.DMA,
        ],
        compiler_params=_SC_PARAMS,
    )
    return kfn(res, tk5, scat5, w16c, nid)


def _gs_comp_body(res_hbm, tk5_hbm, scat5_hbm, w16c_hbm, nid_hbm,
                  g5f_hbm, w16d_hbm,
                  tk_v, sc_v, nid_v, rows_v, w_v, sem, sem2, sem3):
    wid = lax.axis_index("s") * NC + lax.axis_index("c")

    def body(b, carry):
        nb = wid * CROWS_PER_W + b * CNODES_B
        rb = nb * 5
        pltpu.sync_copy(tk5_hbm.at[pl.ds(rb, CNODES_B * 5)], tk_v)
        pltpu.sync_copy(scat5_hbm.at[pl.ds(rb, CNODES_B * 5)], sc_v)
        pltpu.sync_copy(nid_hbm.at[pl.ds(nb, CNODES_B)], nid_v)
        pltpu.sync_copy(w16c_hbm.at[pl.ds(nb, CNODES_B)], w_v)
        pltpu.async_copy(res_hbm.at[tk_v], rows_v, sem).wait()
        pltpu.async_copy(rows_v, g5f_hbm.at[sc_v], sem2).wait()
        pltpu.async_copy(w_v, w16d_hbm.at[nid_v], sem3).wait()
        return carry

    lax.fori_loop(0, CNB, body, 0)


def _knn_body(x_ref, z_ref, idx_ref, w_ref, tv_s, ti_s):
    j = pl.program_id(1)
    x = x_ref[...]
    z = z_ref[...]
    tn = x.shape[0]
    xden = jnp.maximum(jnp.sqrt(jnp.sum(x * x, axis=1, keepdims=True)), 1e-12)
    xb = (x / xden).astype(jnp.bfloat16)
    zden = jnp.maximum(jnp.sqrt(jnp.sum(z * z, axis=1, keepdims=True)), 1e-12)
    zb = (z / zden).astype(jnp.bfloat16)
    s = lax.dot_general(xb, zb, (((1,), (1,)), ((), ())),
                        preferred_element_type=jnp.float32)
    gcol = j * TK + lax.broadcasted_iota(jnp.int32, (tn, TK), 1)
    s = jnp.where(gcol < NRES, s, -jnp.inf)

    @pl.when(j == 0)
    def _():
        tv_s[...] = jnp.full((tn, 5), -jnp.inf, jnp.float32)
        ti_s[...] = jnp.zeros((tn, 5), jnp.int32)

    old_v = tv_s[...]
    old_i = ti_s[...]
    aug = jnp.concatenate([old_v, s], axis=1)
    aw = 5 + TK
    iota = lax.broadcasted_iota(jnp.int32, (tn, aw), 1)
    iota5 = lax.broadcasted_iota(jnp.int32, (tn, 5), 1)
    nv, ni = [], []
    for _ in range(5):
        m = jnp.max(aug, axis=1, keepdims=True)
        cand = jnp.where(aug == m, iota, aw)
        am = jnp.min(cand, axis=1, keepdims=True)
        old_at = jnp.sum(jnp.where(iota5 == am, old_i, 0), axis=1,
                         keepdims=True)
        gidx = jnp.where(am < 5, old_at, j * TK + (am - 5))
        nv.append(m)
        ni.append(gidx)
        aug = jnp.where(iota == am, -jnp.inf, aug)
    tv = jnp.concatenate(nv, axis=1)
    ti = jnp.concatenate(ni, axis=1)
    tv_s[...] = tv
    ti_s[...] = ti

    @pl.when(j == RK - 1)
    def _():
        tsc = tv / 0.2
        e = jnp.exp(tsc - jnp.max(tsc, axis=1, keepdims=True))
        w_ref[...] = e / jnp.sum(e, axis=1, keepdims=True)
        idx_ref[...] = ti


def _tc_knn(x, res, tn):
    n = x.shape[0]
    return pl.pallas_call(
        _knn_body,
        grid=(n // tn, RK),
        in_specs=[
            pl.BlockSpec((tn, HID), lambda i, j: (i, 0)),
            pl.BlockSpec((TK, HID), lambda i, j: (j, 0)),
        ],
        out_specs=[
            pl.BlockSpec((tn, 5), lambda i, j: (i, 0)),
            pl.BlockSpec((tn, 5), lambda i, j: (i, 0)),
        ],
        out_shape=[
            jax.ShapeDtypeStruct((n, 5), jnp.int32),
            jax.ShapeDtypeStruct((n, 5), jnp.float32),
        ],
        scratch_shapes=[
            pltpu.VMEM((tn, 5), jnp.float32),
            pltpu.VMEM((tn, 5), jnp.int32),
        ],
    )(x, res)


def _final_body(fa_ref, fb_ref, id_ref, od_ref, g_ref, w_ref,
                w1_ref, b1_ref, w2_ref, b2_ref, o_ref):
    ind = id_ref[...][:, 0:1]
    outd = od_ref[...][:, 0:1]
    fsum = jnp.concatenate([fa_ref[...], fb_ref[...]], axis=1)
    feat = fsum / jnp.maximum(ind, 1.0)
    w = w_ref[...]
    g = g_ref[...]
    ze = w[:, 0:1] * g[:, 0:HID]
    for k in range(1, 5):
        ze = ze + w[:, k:k + 1] * g[:, k * HID:(k + 1) * HID]
    feat = jnp.where((ind + outd) == 0.0, ze, feat)
    fb = feat.astype(jnp.bfloat16)
    w1b = w1_ref[...].astype(jnp.bfloat16)
    h = lax.dot_general(fb, w1b, (((1,), (1,)), ((), ())),
                        preferred_element_type=jnp.float32) + b1_ref[...]
    h = jnp.maximum(h, 0.0)
    hb = h.astype(jnp.bfloat16)
    w2b = w2_ref[...].astype(jnp.bfloat16)
    o_ref[...] = lax.dot_general(hb, w2b, (((1,), (1,)), ((), ())),
                                 preferred_element_type=jnp.float32) + b2_ref[...]


def _tc_final(fa, fb, indeg, outdeg, g5, w8, W1, b1, W2, b2):
    return pl.pallas_call(
        _final_body,
        grid=(RN,),
        in_specs=[
            pl.BlockSpec((TN, HHALF), lambda i: (i, 0)),
            pl.BlockSpec((TN, HHALF), lambda i: (i, 0)),
            pl.BlockSpec((TN, 16), lambda i: (i, 0)),
            pl.BlockSpec((TN, 16), lambda i: (i, 0)),
            pl.BlockSpec((TN, 5 * HID), lambda i: (i, 0)),
            pl.BlockSpec((TN, 16), lambda i: (i, 0)),
            pl.BlockSpec((HID, HID), lambda i: (0, 0)),
            pl.BlockSpec((1, HID), lambda i: (0, 0)),
            pl.BlockSpec((HID, HID), lambda i: (0, 0)),
            pl.BlockSpec((1, HID), lambda i: (0, 0)),
        ],
        out_specs=pl.BlockSpec((TN, HID), lambda i: (i, 0)),
        out_shape=jax.ShapeDtypeStruct((N_NODES, HID), jnp.float32),
    )(fa, fb, indeg, outdeg, g5, w8, W1, b1, W2, b2)


def kernel(edge_index, etypes, inv, ent_type_feat_emb,
           rel_head_emb, rel_tail_emb, res_ent_emb, W1, b1, W2, b2):
    ei = edge_index.astype(jnp.int32)
    et = etypes.astype(jnp.int32)
    iv = inv.astype(jnp.int32)

    t = jnp.concatenate([rel_tail_emb, rel_head_emb], axis=0)
    t2 = jnp.pad(t, ((0, TROWS - 2 * NREL), (0, 0)))
    ta = t2[:, :HHALF]
    tb = t2[:, HHALF:]
    zf = jnp.zeros((STRIPE, HHALF), jnp.float32)
    zd = jnp.zeros((ROWS_PER_SUB, 16), jnp.float32)

    indeg, outdeg = _sc_degrees(ei[1], ei[0], zd)
    featp = _sc_edge_aggregate(et, iv, ei[1], ta, tb, zf)

    ent = ent_type_feat_emb
    res = res_ent_emb
    zmask = (indeg[:N_NODES, 0] + outdeg[:N_NODES, 0]) == 0.0
    z_count = jnp.sum(zmask.astype(jnp.int32))
    pos = jnp.cumsum(zmask.astype(jnp.int32)) - 1
    scat_pos = jnp.where(zmask, pos, P_COMP)
    comp_idx = (jnp.zeros((P_COMP,), jnp.int32)
                .at[scat_pos].set(jnp.arange(N_NODES, dtype=jnp.int32),
                                  mode="drop"))

    def comp_branch(_):
        xc = _sc_gather_xcomp(ent, comp_idx)
        tki, wc = _tc_knn(xc, res, P_COMP)
        tk5 = tki.reshape(-1)
        scat5 = (comp_idx[:, None] * 5
                 + jnp.arange(5, dtype=jnp.int32)[None, :]).reshape(-1)
        w16c = jnp.pad(wc, ((0, 0), (0, 11)))
        g5f, w16d = _sc_gather_scatter_comp(res, tk5, scat5, w16c, comp_idx)
        return g5f.reshape(NPAD, 5 * HID), w16d

    def dense_branch(_):
        tki, w = _tc_knn(ent, res, TN)
        idxf = jnp.pad(tki, ((0, NPAD - N_NODES), (0, 0))).reshape(-1)
        gath = _sc_gather_rows(res, idxf)
        w16 = jnp.pad(w, ((0, NPAD - N_NODES), (0, 11)))
        return gath.reshape(NPAD, 5 * HID), w16

    g5, w16d = lax.cond(z_count <= P_COMP, comp_branch, dense_branch, None)

    return _tc_final(featp[0], featp[1], indeg, outdeg, g5, w16d,
                     W1, b1.reshape(1, HID), W2, b2.reshape(1, HID))
